# Initial kernel scaffold; baseline (speedup 1.0000x reference)
import jax
import jax.numpy as jnp
from jax.experimental import pallas as pl


def kernel(event_name, level, fqid, room_fqid, W_event_name, W_level, W_fqid, W_room_fqid):
    raise NotImplementedError("write your pallas kernel here")


# speedup vs baseline: 5.7415x; 5.7415x over previous
import functools

import jax
import jax.numpy as jnp
from jax import lax
from jax.experimental import pallas as pl
from jax.experimental.pallas import tpu as pltpu
from jax.experimental.pallas import tpu_sc as plsc

B, L, D = 4096, 200, 16
EVENT_N, LEVEL_N, FQID_N, ROOM_N = 1000, 1000, 100000, 1000

NROWS = 4 * B * L
IDX_MINOR = 128
IDX_ROWS = NROWS // IDX_MINOR
NC, NS = 2, 16
NW = NC * NS
ROWS_PER_W = IDX_ROWS // NW
KC = 16
G = ROWS_PER_W // KC

_mesh = plsc.VectorSubcoreMesh(core_axis_name="c", subcore_axis_name="s")


@functools.partial(
    pl.kernel,
    out_type=jax.ShapeDtypeStruct((NROWS, D), jnp.float32),
    mesh=_mesh,
    compiler_params=pltpu.CompilerParams(use_tc_tiling_on_sc=False),
    scratch_types=[
        pltpu.VMEM((KC, IDX_MINOR), jnp.int32),
        pltpu.VMEM((KC * IDX_MINOR, D), jnp.float32),
        pltpu.SemaphoreType.DMA,
    ],
)
def _gather_all(table_hbm, idx_hbm, out_hbm, idx_v, rows_v, sem):
    wid = lax.axis_index("s") * NC + lax.axis_index("c")
    row0 = wid * ROWS_PER_W

    def body(g, carry):
        base = row0 + g * KC
        pltpu.sync_copy(idx_hbm.at[pl.ds(base, KC)], idx_v)
        copies = [
            pltpu.async_copy(
                table_hbm.at[idx_v.at[j]],
                rows_v.at[pl.ds(j * IDX_MINOR, IDX_MINOR)],
                sem,
            )
            for j in range(KC)
        ]
        for c in copies:
            c.wait()
        pltpu.sync_copy(rows_v, out_hbm.at[pl.ds(base * IDX_MINOR, KC * IDX_MINOR)])
        return carry

    lax.fori_loop(0, G, body, 0)


def kernel(event_name, level, fqid, room_fqid, W_event_name, W_level, W_fqid, W_room_fqid):
    table = jnp.concatenate([W_event_name, W_level, W_fqid, W_room_fqid], axis=0)
    idx = jnp.stack(
        [
            event_name.astype(jnp.int32),
            level.astype(jnp.int32) + EVENT_N,
            fqid.astype(jnp.int32) + (EVENT_N + LEVEL_N),
            room_fqid.astype(jnp.int32) + (EVENT_N + LEVEL_N + FQID_N),
        ],
        axis=-1,
    ).reshape(IDX_ROWS, IDX_MINOR)
    out = _gather_all(table, idx)
    return out.reshape(B, L, 4 * D)


# speedup vs baseline: 6.8992x; 1.2016x over previous
import functools

import jax
import jax.numpy as jnp
from jax import lax
from jax.experimental import pallas as pl
from jax.experimental.pallas import tpu as pltpu
from jax.experimental.pallas import tpu_sc as plsc

B, L, D = 4096, 200, 16
NT = 4
NC, NS = 2, 16
NW = NC * NS
BPW = B // NW
NL = 4
NCHUNK = L // NL

_mesh = plsc.VectorSubcoreMesh(core_axis_name="c", subcore_axis_name="s")


@functools.partial(
    pl.kernel,
    out_type=jax.ShapeDtypeStruct((L, NT * D, B), jnp.float32),
    mesh=_mesh,
    compiler_params=pltpu.CompilerParams(
        use_tc_tiling_on_sc=False, needs_layout_passes=False
    ),
    scratch_types=[
        pltpu.VMEM((NT, NL, BPW), jnp.int32),
        pltpu.VMEM((NL * NT * BPW, D), jnp.float32),
        pltpu.VMEM((NL, NT * D, BPW), jnp.float32),
        pltpu.SemaphoreType.DMA,
    ],
)
def _embed_kernel(w0, w1, w2, w3, i0, i1, i2, i3, out_hbm, idx_v, stage_v, outbuf_v, sem):
    tables = (w0, w1, w2, w3)
    idxs = (i0, i1, i2, i3)
    wid = lax.axis_index("s") * NC + lax.axis_index("c")
    b0 = wid * BPW
    lane = lax.iota(jnp.int32, 16)
    d_idx = [lane + t * D for t in range(NT)]

    def chunk(c, carry):
        l0 = c * NL
        for t in range(NT):
            pltpu.sync_copy(idxs[t].at[pl.ds(l0, NL), pl.ds(b0, BPW)], idx_v.at[t])
        copies = []
        for l in range(NL):
            for t in range(NT):
                copies.append(
                    pltpu.async_copy(
                        tables[t].at[idx_v.at[t, l]],
                        stage_v.at[pl.ds((l * NT + t) * BPW, BPW)],
                        sem,
                    )
                )
        for cp in copies:
            cp.wait()

        def transpose_bb(bb, inner_carry):
            bvec = jnp.full((16,), 0, jnp.int32) + bb
            for l in range(NL):
                lvec = jnp.full((16,), l, jnp.int32)
                for t in range(NT):
                    row = (l * NT + t) * BPW + bb
                    v = stage_v[row, :]
                    plsc.store_scatter(outbuf_v, [lvec, d_idx[t], bvec], v)
            return inner_carry

        lax.fori_loop(0, BPW, transpose_bb, 0)
        for l in range(NL):
            pltpu.sync_copy(outbuf_v.at[l], out_hbm.at[l0 + l, :, pl.ds(b0, BPW)])
        return carry

    lax.fori_loop(0, NCHUNK, chunk, 0)


def kernel(event_name, level, fqid, room_fqid, W_event_name, W_level, W_fqid, W_room_fqid):
    iT = [a.astype(jnp.int32).T for a in (event_name, level, fqid, room_fqid)]
    out = _embed_kernel(W_event_name, W_level, W_fqid, W_room_fqid, *iT)
    return out.transpose(2, 0, 1)


# speedup vs baseline: 7.7631x; 1.1252x over previous
import functools

import jax
import jax.numpy as jnp
from jax import lax
from jax.experimental import pallas as pl
from jax.experimental.pallas import tpu as pltpu
from jax.experimental.pallas import tpu_sc as plsc

B, L, D = 4096, 200, 16
NT = 4
NC, NS = 2, 16
NW = NC * NS
BPW = B // NW
NL = 4
NCHUNK = L // NL
ROWS = NL * NT * BPW
UB = 4

_mesh = plsc.VectorSubcoreMesh(core_axis_name="c", subcore_axis_name="s")


@functools.partial(
    pl.kernel,
    out_type=jax.ShapeDtypeStruct((L, NT * D, B), jnp.float32),
    mesh=_mesh,
    compiler_params=pltpu.CompilerParams(
        use_tc_tiling_on_sc=False, needs_layout_passes=False
    ),
    scratch_types=[
        pltpu.VMEM((2, NT, NL, BPW), jnp.int32),
        pltpu.VMEM((2 * ROWS, D), jnp.float32),
        pltpu.VMEM((NL, NT * D, BPW), jnp.float32),
        pltpu.SemaphoreType.DMA,
        pltpu.SemaphoreType.DMA,
    ],
)
def _embed_kernel(w0, w1, w2, w3, i0, i1, i2, i3, out_hbm, idx_v, stage_v, outbuf_v, sem0, sem1):
    tables = (w0, w1, w2, w3)
    idxs = (i0, i1, i2, i3)
    sems = (sem0, sem1)
    wid = lax.axis_index("s") * NC + lax.axis_index("c")
    b0 = wid * BPW
    lane = lax.iota(jnp.int32, 16)
    d_idx = [lane + t * D for t in range(NT)]

    def fetch(c, slot):
        l0 = c * NL
        for t in range(NT):
            pltpu.sync_copy(idxs[t].at[pl.ds(l0, NL), pl.ds(b0, BPW)], idx_v.at[slot, t])
        for l in range(NL):
            for t in range(NT):
                pltpu.async_copy(
                    tables[t].at[idx_v.at[slot, t, l]],
                    stage_v.at[pl.ds(slot * ROWS + (l * NT + t) * BPW, BPW)],
                    sems[slot],
                )

    def drain(slot):
        pltpu.make_async_copy(
            w2.at[pl.ds(0, ROWS)], stage_v.at[pl.ds(slot * ROWS, ROWS)], sems[slot]
        ).wait()

    def process(c, slot):
        base = slot * ROWS

        def tbody(i, carry):
            for u in range(UB):
                bb = i * UB + u
                bvec = jnp.full((16,), 0, jnp.int32) + bb
                for l in range(NL):
                    lvec = jnp.full((16,), l, jnp.int32)
                    for t in range(NT):
                        v = stage_v[base + (l * NT + t) * BPW + bb, :]
                        plsc.store_scatter(outbuf_v, [lvec, d_idx[t], bvec], v)
            return carry

        lax.fori_loop(0, BPW // UB, tbody, 0)
        l0 = c * NL
        for l in range(NL):
            pltpu.sync_copy(outbuf_v.at[l], out_hbm.at[l0 + l, :, pl.ds(b0, BPW)])

    fetch(0, 0)

    def body(k, carry):
        c0 = 2 * k
        fetch(c0 + 1, 1)
        drain(0)
        process(c0, 0)

        @pl.when(c0 + 2 < NCHUNK)
        def _():
            fetch(c0 + 2, 0)

        drain(1)
        process(c0 + 1, 1)
        return carry

    lax.fori_loop(0, NCHUNK // 2, body, 0)


def kernel(event_name, level, fqid, room_fqid, W_event_name, W_level, W_fqid, W_room_fqid):
    iT = [a.astype(jnp.int32).T for a in (event_name, level, fqid, room_fqid)]
    out = _embed_kernel(W_event_name, W_level, W_fqid, W_room_fqid, *iT)
    return out.transpose(2, 0, 1)


# speedup vs baseline: 9.0977x; 1.1719x over previous
import functools

import jax
import jax.numpy as jnp
from jax import lax
from jax.experimental import pallas as pl
from jax.experimental.pallas import tpu as pltpu
from jax.experimental.pallas import tpu_sc as plsc

B, L, D = 4096, 200, 16
NT = 4
NC, NS = 2, 16
NW = NC * NS
BPW = B // NW
NL = 4
NCHUNK = L // NL
ROWS = NL * NT * BPW
UB = 8

_mesh = plsc.VectorSubcoreMesh(core_axis_name="c", subcore_axis_name="s")


@functools.partial(
    pl.kernel,
    out_type=jax.ShapeDtypeStruct((L, NT * D, B), jnp.float32),
    mesh=_mesh,
    compiler_params=pltpu.CompilerParams(
        use_tc_tiling_on_sc=False, needs_layout_passes=False
    ),
    scratch_types=[
        pltpu.VMEM((2, NT, NL, BPW), jnp.int32),
        pltpu.VMEM((2 * ROWS, D), jnp.float32),
        pltpu.VMEM((NL, NT * D, BPW), jnp.float32),
        pltpu.SemaphoreType.DMA,
        pltpu.SemaphoreType.DMA,
    ],
)
def _embed_kernel(w0, w1, w2, w3, i0, i1, i2, i3, out_hbm, idx_v, stage_v, outbuf_v, sem0, sem1):
    tables = (w0, w1, w2, w3)
    idxs = (i0, i1, i2, i3)
    sems = (sem0, sem1)
    wid = lax.axis_index("s") * NC + lax.axis_index("c")
    b0 = wid * BPW
    lane = lax.iota(jnp.int32, 16)
    d_idx = [lane + t * D for t in range(NT)]

    def fetch(c, slot):
        l0 = c * NL
        for t in range(NT):
            pltpu.sync_copy(idxs[t].at[pl.ds(l0, NL), pl.ds(b0, BPW)], idx_v.at[slot, t])
        for l in range(NL):
            for t in range(NT):
                pltpu.async_copy(
                    tables[t].at[idx_v.at[slot, t, l]],
                    stage_v.at[pl.ds(slot * ROWS + (l * NT + t) * BPW, BPW)],
                    sems[slot],
                )

    def drain(slot):
        pltpu.make_async_copy(
            w2.at[pl.ds(0, ROWS)], stage_v.at[pl.ds(slot * ROWS, ROWS)], sems[slot]
        ).wait()

    def process(c, slot):
        base = slot * ROWS

        @plsc.parallel_loop(0, BPW, 1, unroll=UB)
        def tbody(bb):
            bvec = jnp.full((16,), 0, jnp.int32) + bb
            for l in range(NL):
                lvec = jnp.full((16,), l, jnp.int32)
                for t in range(NT):
                    v = stage_v[base + (l * NT + t) * BPW + bb, :]
                    plsc.store_scatter(outbuf_v, [lvec, d_idx[t], bvec], v)
        l0 = c * NL
        for l in range(NL):
            pltpu.sync_copy(outbuf_v.at[l], out_hbm.at[l0 + l, :, pl.ds(b0, BPW)])

    fetch(0, 0)

    def body(k, carry):
        c0 = 2 * k
        fetch(c0 + 1, 1)
        drain(0)
        process(c0, 0)

        @pl.when(c0 + 2 < NCHUNK)
        def _():
            fetch(c0 + 2, 0)

        drain(1)
        process(c0 + 1, 1)
        return carry

    lax.fori_loop(0, NCHUNK // 2, body, 0)


def kernel(event_name, level, fqid, room_fqid, W_event_name, W_level, W_fqid, W_room_fqid):
    iT = [a.astype(jnp.int32).T for a in (event_name, level, fqid, room_fqid)]
    out = _embed_kernel(W_event_name, W_level, W_fqid, W_room_fqid, *iT)
    return out.transpose(2, 0, 1)
